# baseline (speedup 1.0000x reference)
import functools

import jax
import jax.numpy as jnp
from jax import lax
from jax.experimental import pallas as pl
from jax.experimental.pallas import tpu as pltpu
from jax.experimental.pallas import tpu_sc as plsc

V = 10000
E = 64
BS = 16384
XMAX = 100.0
ALPHA = 0.75

NC = 2
NS = 16
L = 16
NW = NC * NS
NSTRIPE = (V + 127) // 128
CH = 64

_LN2 = 0.6931471805599453
_LNXMAX = 4.605170185988092


def _vlog(x):
    bits = plsc.bitcast(x, jnp.int32)
    e = ((bits >> 23) & 255) - 127
    m = plsc.bitcast((bits & 0x007FFFFF) | 0x3F800000, jnp.float32)
    t = (m - 1.0) / (m + 1.0)
    t2 = t * t
    lnm = 2.0 * t * (1.0 + t2 * (1.0 / 3 + t2 * (1.0 / 5 + t2 * (1.0 / 7))))
    return e.astype(jnp.float32) * _LN2 + lnm


def _sc_body(word_h, ctx_h, tab_h, bw_h, bc_h, comat_h, out_h,
             word_v, ctx_v, stripe_v, widx_v, cidx_v,
             arow, brow, srow, bw_v, bc_v, out_v, sem):
    wid = lax.axis_index("s") * NC + lax.axis_index("c")
    ns = jnp.where(wid < 15, 3, 2)
    k0 = jnp.where(wid < 15, 3 * wid, 2 * wid + 15)

    pltpu.sync_copy(word_h, word_v)
    pltpu.sync_copy(ctx_h, ctx_v)
    pltpu.sync_copy(bw_h, bw_v)
    pltpu.sync_copy(bc_h, bc_v)

    lane = lax.iota(jnp.int32, L)

    def coarse(i, off):
        ws = word_v[pl.ds(i * L, L)]
        cs = ctx_v[pl.ds(i * L, L)]
        t = (cs >> 7) - k0
        m = t.astype(jnp.uint32) < ns.astype(jnp.uint32)
        packed = (ws << 14) | cs
        plsc.store_compressed(word_v.at[pl.ds(off, L)], packed, mask=m)
        return off + plsc.all_reduce_population_count(m)[0]

    n1 = lax.fori_loop(0, BS // L, coarse, 0)

    acc = jnp.zeros((L,), jnp.float32)

    def stripe_body(s, acc_s):
        k = k0 + s
        kb = pl.multiple_of(k << 7, 128)
        stripe = comat_h.at[:, pl.ds(kb, 128)]

        def fine(i, off):
            p = word_v[pl.ds(i * L, L)]
            m = (((p & 16383) >> 7) == k) & ((i * L + lane) < n1)
            plsc.store_compressed(stripe_v.at[pl.ds(off, L)], p, mask=m)
            return off + plsc.all_reduce_population_count(m)[0]

        n2 = lax.fori_loop(0, (n1 + L - 1) // L, fine, 0)
        for j in range(CH // L):
            stripe_v[pl.ds(n2 + j * L, L)] = jnp.zeros((L,), jnp.int32)

        def unpack_fire(c, slot):
            for g in range(CH // L):
                p16 = stripe_v[pl.ds(c * CH + g * L, L)]
                widx_v[slot, pl.ds(g * L, L)] = p16 >> 14
                cidx_v[slot, pl.ds(g * L, L)] = p16 & 16383
            pltpu.async_copy(stripe.at[widx_v.at[slot]], srow.at[slot], sem)
            pltpu.async_copy(tab_h.at[widx_v.at[slot]], arow.at[slot], sem)
            pltpu.async_copy(tab_h.at[cidx_v.at[slot]], brow.at[slot], sem)

        def drain(slot):
            pltpu.make_async_copy(stripe.at[widx_v.at[slot]],
                                  srow.at[slot], sem).wait()
            pltpu.make_async_copy(tab_h.at[widx_v.at[slot]],
                                  arow.at[slot], sem).wait()
            pltpu.make_async_copy(tab_h.at[cidx_v.at[slot]],
                                  brow.at[slot], sem).wait()

        def compute(c, slot, acc_c):
            sv = jnp.full((L,), slot, jnp.int32)

            def group(g, acc_g):
                rid = g * L + lane
                w16 = widx_v[slot, pl.ds(g * L, L)]
                c16 = cidx_v[slot, pl.ds(g * L, L)]
                dot = jnp.zeros((L,), jnp.float32)
                for e in range(E):
                    ev = jnp.full((L,), e, jnp.int32)
                    wv = plsc.load_gather(arow, [sv, rid, ev])
                    cv = plsc.load_gather(brow, [sv, rid, ev + E])
                    dot = dot + wv * cv
                bwg = plsc.load_gather(bw_v, [w16])
                bcg = plsc.load_gather(bc_v, [c16])
                co = plsc.load_gather(srow, [sv, rid, c16 & 127])
                lnco = _vlog(co)
                wgt = jnp.where(co < XMAX,
                                jnp.exp(ALPHA * (lnco - _LNXMAX)),
                                jnp.ones_like(co))
                d = dot + bwg + bcg - lnco
                valid = (c * CH + g * L + lane) < n2
                return acc_g + jnp.where(valid, d * d * wgt, 0.0)

            return lax.fori_loop(0, CH // L, group, acc_c)

        nch = (n2 + CH - 1) // CH

        @pl.when(nch > 0)
        def _():
            unpack_fire(0, 0)

        def chunk_pair(p, acc_p):
            ca = 2 * p

            @pl.when(ca + 1 < nch)
            def _():
                unpack_fire(ca + 1, 1)

            drain(0)
            acc_p = compute(ca, 0, acc_p)

            @pl.when(ca + 2 < nch)
            def _():
                unpack_fire(ca + 2, 0)

            def do_b(acc_b):
                drain(1)
                return compute(ca + 1, 1, acc_b)

            return lax.cond(ca + 1 < nch, do_b, lambda a: a, acc_p)

        return lax.fori_loop(0, (nch + 1) // 2, chunk_pair, acc_s)

    acc = lax.fori_loop(0, ns, stripe_body, acc)

    out_v[pl.ds(0, L)] = acc
    pltpu.sync_copy(out_v, out_h.at[pl.ds(wid * L, L)])


_sc_loss = functools.partial(
    pl.kernel,
    out_type=jax.ShapeDtypeStruct((NW * L,), jnp.float32),
    mesh=plsc.VectorSubcoreMesh(core_axis_name="c", subcore_axis_name="s",
                                num_cores=NC, num_subcores=NS),
    compiler_params=pltpu.CompilerParams(needs_layout_passes=False),
    scratch_types=[
        pltpu.VMEM((BS,), jnp.int32),
        pltpu.VMEM((BS,), jnp.int32),
        pltpu.VMEM((BS + CH,), jnp.int32),
        pltpu.VMEM((2, CH), jnp.int32),
        pltpu.VMEM((2, CH), jnp.int32),
        pltpu.VMEM((2, CH, 2 * E), jnp.float32),
        pltpu.VMEM((2, CH, 2 * E), jnp.float32),
        pltpu.VMEM((2, CH, 128), jnp.float32),
        pltpu.VMEM((V,), jnp.float32),
        pltpu.VMEM((V,), jnp.float32),
        pltpu.VMEM((L,), jnp.float32),
        pltpu.SemaphoreType.DMA,
    ],
)(_sc_body)


def kernel(word, context, Wword, Wctx, bword, bctx, comat):
    word = word.astype(jnp.int32)
    context = context.astype(jnp.int32)
    table = jnp.concatenate([Wword, Wctx], axis=1)
    parts = _sc_loss(word, context, table,
                     bword.reshape(-1), bctx.reshape(-1), comat)
    return jnp.sum(parts)


# speedup vs baseline: 3.6093x; 3.6093x over previous
import functools

import jax
import jax.numpy as jnp
from jax import lax
from jax.experimental import pallas as pl
from jax.experimental.pallas import tpu as pltpu
from jax.experimental.pallas import tpu_sc as plsc

V = 10000
E = 64
BS = 16384
XMAX = 100.0
ALPHA = 0.75

NC = 2
NS = 16
L = 16
NW = NC * NS
BPW = BS // NW
HALF = BPW // 2
CH = 128
WV = 32
NWAVE = HALF // WV

_LN2 = 0.6931471805599453
_LNXMAX = 4.605170185988092


def _vlog(x):
    bits = plsc.bitcast(x, jnp.int32)
    e = ((bits >> 23) & 255) - 127
    m = plsc.bitcast((bits & 0x007FFFFF) | 0x3F800000, jnp.float32)
    t = (m - 1.0) / (m + 1.0)
    t2 = t * t
    lnm = 2.0 * t * (1.0 + t2 * (1.0 / 3 + t2 * (1.0 / 5 + t2 * (1.0 / 7))))
    return e.astype(jnp.float32) * _LN2 + lnm


def _sc_body(word_h, ctx_h, tab_h, bw_h, bc_h, comat_h, out_h,
             word_v, ctx_v, bw_v, bc_v, wrows, crows, co_v,
             strip_a, strip_b, idx_a, idx_b, out_v, sem, semr):
    wid = lax.axis_index("s") * NC + lax.axis_index("c")
    base = wid * BPW

    pltpu.sync_copy(word_h.at[pl.ds(base, BPW)], word_v)
    pltpu.sync_copy(ctx_h.at[pl.ds(base, BPW)], ctx_v)
    pltpu.sync_copy(bw_h, bw_v)
    pltpu.sync_copy(bc_h, bc_v)

    lane = lax.iota(jnp.int32, L)
    zero = jnp.zeros((L,), jnp.int32)
    acc = jnp.zeros((L,), jnp.float32)

    for half in range(2):
        hb = half * HALF
        row_copies = []
        for j in range(HALF // CH):
            sl = pl.ds(hb + j * CH, CH)
            dsl = pl.ds(j * CH, CH)
            row_copies.append(
                pltpu.async_copy(tab_h.at[word_v.at[sl]], wrows.at[dsl], semr))
            row_copies.append(
                pltpu.async_copy(tab_h.at[ctx_v.at[sl]], crows.at[dsl], semr))

        def wave_pair(p, carry):
            def fire(buf, ibuf, w):
                wbase = hb + w * WV
                css = []
                cps = []
                for g in range(WV // L):
                    ws = word_v[pl.ds(wbase + g * L, L)]
                    plsc.store_scatter(ibuf, [(lane + g * L) * 8], ws)
                    cs = ctx_v[pl.ds(wbase + g * L, L)]
                    css.append(cs)
                    for j in range(L):
                        cb = pl.multiple_of((cs[j] >> 7) << 7, 128)
                        stripe = comat_h.at[:, pl.ds(cb, 128)]
                        idx = ibuf.at[pl.ds((g * L + j) * 8, 1)]
                        cps.append(pltpu.async_copy(
                            stripe.at[idx], buf.at[g * L + j], sem))
                return css, cps

            def extract(buf, css, cps, w):
                for cp in cps:
                    cp.wait()
                for g in range(WV // L):
                    gv = jnp.full((L,), g * L, jnp.int32) + lane
                    co_v[pl.ds(hb + w * WV + g * L, L)] = plsc.load_gather(
                        buf, [gv, zero, css[g] & 127])

            wa = p * 2
            csa, cpsa = fire(strip_a, idx_a, wa)
            csb, cpsb = fire(strip_b, idx_b, wa + 1)
            extract(strip_a, csa, cpsa, wa)
            extract(strip_b, csb, cpsb, wa + 1)
            return carry

        lax.fori_loop(0, NWAVE // 2, wave_pair, 0)

        for cp in row_copies:
            cp.wait()

        def group(g, acc_in):
            rid = g * L + lane
            dot = jnp.zeros((L,), jnp.float32)
            for e in range(E):
                ev = jnp.full((L,), e, jnp.int32)
                wv = plsc.load_gather(wrows, [rid, ev])
                cv = plsc.load_gather(crows, [rid, ev + E])
                dot = dot + wv * cv
            sl = pl.ds(hb + g * L, L)
            bwg = plsc.load_gather(bw_v, [word_v[sl]])
            bcg = plsc.load_gather(bc_v, [ctx_v[sl]])
            co = co_v[sl]
            lnco = _vlog(co)
            wgt = jnp.where(co < XMAX,
                            jnp.exp(ALPHA * (lnco - _LNXMAX)),
                            jnp.ones_like(co))
            d = dot + bwg + bcg - lnco
            return acc_in + d * d * wgt

        acc = lax.fori_loop(0, HALF // L, group, acc)

    out_v[pl.ds(0, L)] = acc
    pltpu.sync_copy(out_v, out_h.at[pl.ds(wid * L, L)])


_sc_loss = functools.partial(
    pl.kernel,
    out_type=jax.ShapeDtypeStruct((NW * L,), jnp.float32),
    mesh=plsc.VectorSubcoreMesh(core_axis_name="c", subcore_axis_name="s",
                                num_cores=NC, num_subcores=NS),
    compiler_params=pltpu.CompilerParams(needs_layout_passes=False),
    scratch_types=[
        pltpu.VMEM((BPW,), jnp.int32),
        pltpu.VMEM((BPW,), jnp.int32),
        pltpu.VMEM((V,), jnp.float32),
        pltpu.VMEM((V,), jnp.float32),
        pltpu.VMEM((HALF, 2 * E), jnp.float32),
        pltpu.VMEM((HALF, 2 * E), jnp.float32),
        pltpu.VMEM((BPW,), jnp.float32),
        pltpu.VMEM((WV, 1, 128), jnp.float32),
        pltpu.VMEM((WV, 1, 128), jnp.float32),
        pltpu.VMEM((WV * 8,), jnp.int32),
        pltpu.VMEM((WV * 8,), jnp.int32),
        pltpu.VMEM((L,), jnp.float32),
        pltpu.SemaphoreType.DMA,
        pltpu.SemaphoreType.DMA,
    ],
)(_sc_body)


def kernel(word, context, Wword, Wctx, bword, bctx, comat):
    word = word.astype(jnp.int32)
    context = context.astype(jnp.int32)
    table = jnp.concatenate([Wword, Wctx], axis=1)
    parts = _sc_loss(word, context, table,
                     bword.reshape(-1), bctx.reshape(-1), comat)
    return jnp.sum(parts)
